# baseline (speedup 1.0000x reference)
import functools

import jax
import jax.numpy as jnp
from jax import lax
from jax.experimental import pallas as pl
from jax.experimental.pallas import tpu as pltpu
from jax.experimental.pallas import tpu_sc as plsc

_B, _N = 8, 8192
_G, _S = 512, 32
_GT = 128
_NW = 32
_WPB = _NW // _B
_GPW = _G // _WPB


def _fps_body(pos_ref, sel_ref, cxo_ref, cyo_ref, czo_ref, d_ref):
    x = pos_ref[0]
    y = pos_ref[1]
    z = pos_ref[2]
    iota = lax.broadcasted_iota(jnp.int32, (_B, _N), 1)
    cx = x[:, 0:1]
    cy = y[:, 0:1]
    cz = z[:, 0:1]
    dx = x - cx
    dy = y - cy
    dz = z - cz
    d_ref[...] = (dx * dx + dy * dy) + dz * dz
    sel_ref[0] = jnp.zeros((_B, 1), jnp.int32)
    cxo_ref[0] = cx
    cyo_ref[0] = cy
    czo_ref[0] = cz

    def body(i, carry):
        d = d_ref[...]
        m = jnp.max(d, axis=1, keepdims=True)
        nxt = jnp.min(jnp.where(d == m, iota, _N), axis=1, keepdims=True)
        onehot = iota == nxt
        cx = jnp.sum(jnp.where(onehot, x, 0.0), axis=1, keepdims=True)
        cy = jnp.sum(jnp.where(onehot, y, 0.0), axis=1, keepdims=True)
        cz = jnp.sum(jnp.where(onehot, z, 0.0), axis=1, keepdims=True)
        sel_ref[pl.ds(i, 1)] = nxt[None]
        cxo_ref[pl.ds(i, 1)] = cx[None]
        cyo_ref[pl.ds(i, 1)] = cy[None]
        czo_ref[pl.ds(i, 1)] = cz[None]
        dx = x - cx
        dy = y - cy
        dz = z - cz
        nd = (dx * dx + dy * dy) + dz * dz
        d_ref[...] = jnp.minimum(d, nd)
        return carry

    lax.fori_loop(1, _G, body, 0)


_fps = pl.pallas_call(
    _fps_body,
    out_shape=(
        jax.ShapeDtypeStruct((_G, _B, 1), jnp.int32),
        jax.ShapeDtypeStruct((_G, _B, 1), jnp.float32),
        jax.ShapeDtypeStruct((_G, _B, 1), jnp.float32),
        jax.ShapeDtypeStruct((_G, _B, 1), jnp.float32),
    ),
    scratch_shapes=[pltpu.VMEM((_B, _N), jnp.float32)],
)


def _knn_body(pos_ref, c_ref, nn_ref, d_ref, nnj_ref):
    pos_m = pos_ref[0]
    cen = c_ref[0]
    x = pos_m[0:1, :]
    y = pos_m[1:2, :]
    z = pos_m[2:3, :]
    cx = cen[:, 0:1]
    cy = cen[:, 1:2]
    cz = cen[:, 2:3]
    b2 = (x * x + y * y) + z * z
    a2 = (cx * cx + cy * cy) + cz * cz
    ab = lax.dot_general(cen, pos_m, (((1,), (0,)), ((), ())),
                         preferred_element_type=jnp.float32)
    dsq = (a2 + b2) - 2.0 * ab
    d0 = jnp.sqrt(jnp.maximum(dsq, 0.0))
    d_ref[...] = d0
    m0 = jnp.min(d0, axis=1, keepdims=True)
    iota = lax.broadcasted_iota(jnp.int32, (_GT, _N), 1)

    def body(j, m):
        d = d_ref[...]
        idx = jnp.min(jnp.where(d == m, iota, _N), axis=1, keepdims=True)
        nnj_ref[pl.ds(j, 1)] = idx[None]
        dn = jnp.where(iota == idx, jnp.float32(jnp.inf), d)
        d_ref[...] = dn
        return jnp.min(dn, axis=1, keepdims=True)

    lax.fori_loop(0, _S, body, m0)
    nn_ref[0] = jnp.transpose(nnj_ref[:, :, 0], (1, 0))


_knn = pl.pallas_call(
    _knn_body,
    grid=(_B, _G // _GT),
    in_specs=[
        pl.BlockSpec((1, 3, _N), lambda b, g: (b, 0, 0)),
        pl.BlockSpec((1, _GT, 3), lambda b, g: (b, g, 0)),
    ],
    out_specs=pl.BlockSpec((1, _GT, _S), lambda b, g: (b, g, 0)),
    out_shape=jax.ShapeDtypeStruct((_B, _G, _S), jnp.int32),
    scratch_shapes=[
        pltpu.VMEM((_GT, _N), jnp.float32),
        pltpu.VMEM((_S, _GT, 1), jnp.int32),
    ],
)


def _gather_body(pc_hbm, rep_hbm, nn_hbm, center_hbm, nb_hbm,
                 pc_v, rep_v, nn_v, cen_v, nb_v):
    wid = lax.axis_index("s") * 2 + lax.axis_index("c")
    b = wid // _WPB
    q = wid % _WPB
    g0 = q * _GPW
    pltpu.sync_copy(pc_hbm.at[b], pc_v)
    pltpu.sync_copy(rep_hbm.at[b, pl.ds(g0, _GPW)], rep_v)
    pltpu.sync_copy(nn_hbm.at[b, pl.ds(g0 * _S, _GPW * _S)], nn_v)
    i16 = lax.iota(jnp.int32, 16)

    for t in range(_GPW // 16):
        base = t * 16
        gidx = rep_v[pl.ds(base, 16)] * 8
        rows = (i16 + base) * 8
        for ch in range(8):
            v = plsc.load_gather(pc_v, [gidx + ch])
            plsc.store_scatter(cen_v, [rows + ch], v)

    def body(t, carry):
        base = t * 16
        nidx = nn_v[pl.ds(base, 16)] * 8
        rows = (i16 + base) * 8
        gvec = lax.shift_right_logical(i16 + base, 5) * 8
        for ch in range(8):
            v = plsc.load_gather(pc_v, [nidx + ch])
            if ch < 3:
                c = plsc.load_gather(cen_v, [gvec + ch])
                v = v - c
            plsc.store_scatter(nb_v, [rows + ch], v)
        return carry

    lax.fori_loop(0, _GPW * _S // 16, body, 0)
    pltpu.sync_copy(cen_v, center_hbm.at[b, pl.ds(g0 * 8, _GPW * 8)])
    pltpu.sync_copy(nb_v, nb_hbm.at[b, pl.ds(g0 * _S * 8, _GPW * _S * 8)])


@functools.cache
def _make_gather():
    return functools.partial(
        pl.kernel,
        mesh=plsc.VectorSubcoreMesh(core_axis_name="c", subcore_axis_name="s"),
        compiler_params=pltpu.CompilerParams(needs_layout_passes=False),
        out_type=(
            jax.ShapeDtypeStruct((_B, _G * 8), jnp.float32),
            jax.ShapeDtypeStruct((_B, _G * _S * 8), jnp.float32),
        ),
        scratch_types=[
            pltpu.VMEM((_N * 8,), jnp.float32),
            pltpu.VMEM((_GPW,), jnp.int32),
            pltpu.VMEM((_GPW * _S,), jnp.int32),
            pltpu.VMEM((_GPW * 8,), jnp.float32),
            pltpu.VMEM((_GPW * _S * 8,), jnp.float32),
        ],
    )(_gather_body)


def kernel(pc):
    pos_t = jnp.transpose(pc[:, :, :3], (2, 0, 1))
    sel, cxo, cyo, czo = _fps(pos_t)
    rep_idx = sel[:, :, 0].T
    c_gt = jnp.stack(
        [cxo[:, :, 0].T, cyo[:, :, 0].T, czo[:, :, 0].T], axis=2)
    pos_bt = jnp.transpose(pos_t, (1, 0, 2))
    nn_idx = _knn(pos_bt, c_gt)
    pc_pad = jnp.concatenate(
        [pc, jnp.zeros((_B, _N, 2), pc.dtype)], axis=-1)
    center8, nb8 = _make_gather()(pc_pad.reshape(_B, _N * 8), rep_idx,
                                  nn_idx.reshape(_B, _G * _S))
    nb = nb8.reshape(_B, _G, _S, 8)[..., :6]
    center = center8.reshape(_B, _G, 8)[..., :6]
    return nb, center, nn_idx


# speedup vs baseline: 1.4941x; 1.4941x over previous
import functools

import jax
import jax.numpy as jnp
from jax import lax
from jax.experimental import pallas as pl
from jax.experimental.pallas import tpu as pltpu
from jax.experimental.pallas import tpu_sc as plsc

_B, _N = 8, 8192
_G, _S = 512, 32
_GT = 128
_NW = 32
_WPB = _NW // _B
_GPW = _G // _WPB


def _fps_body(pos_ref, sel_ref, cxo_ref, cyo_ref, czo_ref, d_ref):
    x = pos_ref[0]
    y = pos_ref[1]
    z = pos_ref[2]
    iota = lax.broadcasted_iota(jnp.int32, (_B, _N), 1)
    cx = x[:, 0:1]
    cy = y[:, 0:1]
    cz = z[:, 0:1]
    dx = x - cx
    dy = y - cy
    dz = z - cz
    d_ref[...] = (dx * dx + dy * dy) + dz * dz
    sel_ref[0] = jnp.zeros((_B, 1), jnp.int32)
    cxo_ref[0] = cx
    cyo_ref[0] = cy
    czo_ref[0] = cz

    def body(i, carry):
        d = d_ref[...]
        m = jnp.max(d, axis=1, keepdims=True)
        nxt = jnp.min(jnp.where(d == m, iota, _N), axis=1, keepdims=True)
        onehot = iota == nxt
        cx = jnp.sum(jnp.where(onehot, x, 0.0), axis=1, keepdims=True)
        cy = jnp.sum(jnp.where(onehot, y, 0.0), axis=1, keepdims=True)
        cz = jnp.sum(jnp.where(onehot, z, 0.0), axis=1, keepdims=True)
        sel_ref[pl.ds(i, 1)] = nxt[None]
        cxo_ref[pl.ds(i, 1)] = cx[None]
        cyo_ref[pl.ds(i, 1)] = cy[None]
        czo_ref[pl.ds(i, 1)] = cz[None]
        dx = x - cx
        dy = y - cy
        dz = z - cz
        nd = (dx * dx + dy * dy) + dz * dz
        d_ref[...] = jnp.minimum(d, nd)
        return carry

    lax.fori_loop(1, _G, body, 0)


_fps = pl.pallas_call(
    _fps_body,
    out_shape=(
        jax.ShapeDtypeStruct((_G, _B, 1), jnp.int32),
        jax.ShapeDtypeStruct((_G, _B, 1), jnp.float32),
        jax.ShapeDtypeStruct((_G, _B, 1), jnp.float32),
        jax.ShapeDtypeStruct((_G, _B, 1), jnp.float32),
    ),
    scratch_shapes=[pltpu.VMEM((_B, _N), jnp.float32)],
)


_NS = 16
_NC = _N // _NS
_INF = float("inf")


def _knn_body(pos_ref, c_ref, nn_ref, d3_ref, nnj_ref,
              h_ref, hi_ref, c1_ref, i1_ref, c2_ref, i2_ref, t_ref):
    pos_m = pos_ref[0]
    cen = c_ref[0]
    x = pos_m[0:1, :]
    y = pos_m[1:2, :]
    z = pos_m[2:3, :]
    cx = cen[:, 0:1]
    cy = cen[:, 1:2]
    cz = cen[:, 2:3]
    b2 = (x * x + y * y) + z * z
    a2 = (cx * cx + cy * cy) + cz * cz
    ab = lax.dot_general(cen, pos_m, (((1,), (0,)), ((), ())),
                         preferred_element_type=jnp.float32)
    dsq = (a2 + b2) - 2.0 * ab
    dall = jnp.sqrt(jnp.maximum(dsq, 0.0))
    for s in range(_NS):
        d3_ref[s] = dall[:, s * _NC:(s + 1) * _NC]

    iotac = lax.broadcasted_iota(jnp.int32, (_GT, _NC), 1)

    def _extract_rank():
        m = d3_ref[0]
        for s in range(1, _NS):
            m = jnp.minimum(m, d3_ref[s])
        idxs = jnp.full((_GT, _NC), _NS, jnp.int32)
        for s in range(_NS - 1, -1, -1):
            idxs = jnp.where(d3_ref[s] == m, s, idxs)
        for s in range(_NS):
            d3_ref[s] = jnp.where(idxs == s, _INF, d3_ref[s])
        return m, idxs * _NC + iotac

    h0, hi0 = _extract_rank()
    h_ref[...] = h0
    hi_ref[...] = hi0
    c1, i1 = _extract_rank()
    c1_ref[...] = c1
    i1_ref[...] = i1
    c2, i2 = _extract_rank()
    c2_ref[...] = c2
    i2_ref[...] = i2
    t_ref[...] = jnp.zeros((_GT, _NC), jnp.int32)

    big = jnp.int32(_N * 2)

    def body(j, carry):
        h = h_ref[...]
        hi = hi_ref[...]
        m = jnp.min(h, axis=1, keepdims=True)
        idxg = jnp.min(jnp.where(h == m, hi, big), axis=1, keepdims=True)
        nnj_ref[pl.ds(j, 1)] = idxg[None]
        win = hi == idxg
        tn = t_ref[...] + win.astype(jnp.int32)
        t_ref[...] = tn
        nh = jnp.where(tn == 1, c1_ref[...],
                       jnp.where(tn == 2, c2_ref[...], _INF))
        nhi = jnp.where(tn == 1, i1_ref[...],
                        jnp.where(tn == 2, i2_ref[...], big))
        h_ref[...] = jnp.where(win, nh, h)
        hi_ref[...] = jnp.where(win, nhi, hi)
        need_row = jnp.max(jnp.where(win & (tn >= 3), 1, 0),
                           axis=1, keepdims=True)
        need = jnp.max(need_row) > 0

        @pl.when(need)
        def _refill():
            c0 = jnp.where(need_row > 0, idxg & (_NC - 1), -1)
            s0 = lax.shift_right_logical(idxg, 9)
            cmask = iotac == c0
            colmin = None
            for s in range(_NS):
                slab = jnp.where(cmask & (s0 == s), _INF, d3_ref[s])
                d3_ref[s] = slab
                colmin = slab if colmin is None else jnp.minimum(colmin, slab)
            rv = jnp.min(jnp.where(cmask, colmin, _INF), axis=1, keepdims=True)
            rs = jnp.full((_GT, 1), _NS, jnp.int32)
            for s in range(_NS - 1, -1, -1):
                hit = jnp.max(jnp.where(cmask & (d3_ref[s] == rv), 1, 0),
                              axis=1, keepdims=True)
                rs = jnp.where(hit > 0, s, rs)
            ri = rs * _NC + c0
            h_ref[...] = jnp.where(cmask, rv, h_ref[...])
            hi_ref[...] = jnp.where(cmask, ri, hi_ref[...])

        return carry

    lax.fori_loop(0, _S, body, 0)
    nn_ref[0] = jnp.transpose(nnj_ref[:, :, 0], (1, 0))


_knn = pl.pallas_call(
    _knn_body,
    grid=(_B, _G // _GT),
    in_specs=[
        pl.BlockSpec((1, 3, _N), lambda b, g: (b, 0, 0)),
        pl.BlockSpec((1, _GT, 3), lambda b, g: (b, g, 0)),
    ],
    out_specs=pl.BlockSpec((1, _GT, _S), lambda b, g: (b, g, 0)),
    out_shape=jax.ShapeDtypeStruct((_B, _G, _S), jnp.int32),
    scratch_shapes=[
        pltpu.VMEM((_NS, _GT, _NC), jnp.float32),
        pltpu.VMEM((_S, _GT, 1), jnp.int32),
        pltpu.VMEM((_GT, _NC), jnp.float32),
        pltpu.VMEM((_GT, _NC), jnp.int32),
        pltpu.VMEM((_GT, _NC), jnp.float32),
        pltpu.VMEM((_GT, _NC), jnp.int32),
        pltpu.VMEM((_GT, _NC), jnp.float32),
        pltpu.VMEM((_GT, _NC), jnp.int32),
        pltpu.VMEM((_GT, _NC), jnp.int32),
    ],
)


def _gather_body(pc_hbm, rep_hbm, nn_hbm, center_hbm, nb_hbm,
                 pc_v, rep_v, nn_v, cen_v, nb_v):
    wid = lax.axis_index("s") * 2 + lax.axis_index("c")
    b = wid // _WPB
    q = wid % _WPB
    g0 = q * _GPW
    pltpu.sync_copy(pc_hbm.at[b], pc_v)
    pltpu.sync_copy(rep_hbm.at[b, pl.ds(g0, _GPW)], rep_v)
    pltpu.sync_copy(nn_hbm.at[b, pl.ds(g0 * _S, _GPW * _S)], nn_v)
    i16 = lax.iota(jnp.int32, 16)

    for t in range(_GPW // 16):
        base = t * 16
        gidx = rep_v[pl.ds(base, 16)] * 8
        rows = (i16 + base) * 8
        for ch in range(8):
            v = plsc.load_gather(pc_v, [gidx + ch])
            plsc.store_scatter(cen_v, [rows + ch], v)

    def body(t, carry):
        base = t * 16
        nidx = nn_v[pl.ds(base, 16)] * 8
        rows = (i16 + base) * 8
        gvec = lax.shift_right_logical(i16 + base, 5) * 8
        for ch in range(8):
            v = plsc.load_gather(pc_v, [nidx + ch])
            if ch < 3:
                c = plsc.load_gather(cen_v, [gvec + ch])
                v = v - c
            plsc.store_scatter(nb_v, [rows + ch], v)
        return carry

    lax.fori_loop(0, _GPW * _S // 16, body, 0)
    pltpu.sync_copy(cen_v, center_hbm.at[b, pl.ds(g0 * 8, _GPW * 8)])
    pltpu.sync_copy(nb_v, nb_hbm.at[b, pl.ds(g0 * _S * 8, _GPW * _S * 8)])


@functools.cache
def _make_gather():
    return functools.partial(
        pl.kernel,
        mesh=plsc.VectorSubcoreMesh(core_axis_name="c", subcore_axis_name="s"),
        compiler_params=pltpu.CompilerParams(needs_layout_passes=False),
        out_type=(
            jax.ShapeDtypeStruct((_B, _G * 8), jnp.float32),
            jax.ShapeDtypeStruct((_B, _G * _S * 8), jnp.float32),
        ),
        scratch_types=[
            pltpu.VMEM((_N * 8,), jnp.float32),
            pltpu.VMEM((_GPW,), jnp.int32),
            pltpu.VMEM((_GPW * _S,), jnp.int32),
            pltpu.VMEM((_GPW * 8,), jnp.float32),
            pltpu.VMEM((_GPW * _S * 8,), jnp.float32),
        ],
    )(_gather_body)


def kernel(pc):
    pos_t = jnp.transpose(pc[:, :, :3], (2, 0, 1))
    sel, cxo, cyo, czo = _fps(pos_t)
    rep_idx = sel[:, :, 0].T
    c_gt = jnp.stack(
        [cxo[:, :, 0].T, cyo[:, :, 0].T, czo[:, :, 0].T], axis=2)
    pos_bt = jnp.transpose(pos_t, (1, 0, 2))
    nn_idx = _knn(pos_bt, c_gt)
    pc_pad = jnp.concatenate(
        [pc, jnp.zeros((_B, _N, 2), pc.dtype)], axis=-1)
    center8, nb8 = _make_gather()(pc_pad.reshape(_B, _N * 8), rep_idx,
                                  nn_idx.reshape(_B, _G * _S))
    nb = nb8.reshape(_B, _G, _S, 8)[..., :6]
    center = center8.reshape(_B, _G, 8)[..., :6]
    return nb, center, nn_idx
